# baseline (speedup 1.0000x reference)
import functools

import jax
import jax.numpy as jnp
from jax import lax
from jax.experimental import pallas as pl
from jax.experimental.pallas import tpu as pltpu
from jax.experimental.pallas import tpu_sc as plsc

BATCH = 4096
HIST = 200
DIM = 64
SIZE = 1000000
HPAD = 208
WPR = 2 * HPAD
NGRP = WPR // 16
NC = 2
NS = 16
NW = NC * NS
BPW = BATCH // NW
LANES = 16
NBUF = 4
RPG = 8
NACC = HIST // RPG


def _encoder_body(x_hbm, tbl_hbm, out_hbm, idx_v, rows_v, out_v, *sems):
    cid = lax.axis_index("c")
    sid = lax.axis_index("s")
    wid = sid * NC + cid
    base = wid * BPW

    pltpu.sync_copy(x_hbm.at[pl.ds(base, BPW)], idx_v)

    def start_gather(b, buf):
        for j in range(NGRP):
            idx_vec = idx_v[b, pl.ds(j * LANES, LANES)]
            pltpu.async_copy(
                tbl_hbm.at[idx_vec],
                rows_v.at[buf, pl.ds(j * LANES, LANES)], sems[buf])

    def drain_gather(buf):
        pltpu.make_async_copy(
            tbl_hbm.at[pl.ds(0, WPR)], rows_v.at[buf], sems[buf]).wait()

    iota = lax.iota(jnp.int32, LANES)
    one_bit = jnp.ones((LANES,), jnp.int32)

    def accumulate(b, buf):
        zero = jnp.zeros((LANES,), jnp.int32)

        def group(g, acc):
            v = rows_v[buf, pl.ds(g * 2 * RPG, 2 * RPG)]
            a0, a1, a2, a3 = acc
            for u in range(RPG):
                w0 = jnp.broadcast_to(v[2 * u], (LANES,))
                w1 = jnp.broadcast_to(v[2 * u + 1], (LANES,))
                a0 = a0 + ((w0 >> iota) & one_bit)
                a1 = a1 + ((w0 >> (iota + 16)) & one_bit)
                a2 = a2 + ((w1 >> iota) & one_bit)
                a3 = a3 + ((w1 >> (iota + 16)) & one_bit)
            return a0, a1, a2, a3

        acc = lax.fori_loop(0, NACC, group, (zero,) * 4)

        half = jnp.full((LANES,), HIST // 2, jnp.int32)
        onef = jnp.full((LANES,), 1.0, jnp.float32)
        for k in range(4):
            out_v[b, pl.ds(k * LANES, LANES)] = jnp.where(
                acc[k] > half, onef, -onef)

    for b in range(NBUF):
        start_gather(b, b)

    def outer(i, _):
        row0 = NBUF * i
        for u in range(NBUF):
            b = row0 + u
            drain_gather(u)
            accumulate(b, u)

            @pl.when(b + NBUF < BPW)
            def _():
                start_gather(b + NBUF, u)
        return 0

    lax.fori_loop(0, BPW // NBUF, outer, 0)

    pltpu.sync_copy(out_v, out_hbm.at[pl.ds(base, BPW)])


@jax.jit
def _encoder(x3, packed):
    mesh = plsc.VectorSubcoreMesh(
        core_axis_name="c", subcore_axis_name="s", num_cores=NC,
        num_subcores=NS)
    return pl.kernel(
        _encoder_body,
        out_type=jax.ShapeDtypeStruct((BATCH, DIM), jnp.float32),
        mesh=mesh,
        scratch_types=[
            pltpu.VMEM((BPW, WPR), jnp.int32),
            pltpu.VMEM((NBUF, WPR), jnp.int32),
            pltpu.VMEM((BPW, DIM), jnp.float32),
        ] + [pltpu.SemaphoreType.DMA] * NBUF,
        compiler_params=pltpu.CompilerParams(use_tc_tiling_on_sc=False),
    )(x3, packed)


def kernel(x, embed_weight):
    bits = (embed_weight > 0).astype(jnp.int32)
    shifts = jnp.arange(32, dtype=jnp.int32)
    w0 = (bits[:, :32] << shifts).sum(axis=1, dtype=jnp.int32)
    w1 = (bits[:, 32:] << shifts).sum(axis=1, dtype=jnp.int32)
    packed = jnp.stack([w0, w1], axis=1).reshape(2 * SIZE)
    xw = jnp.stack([2 * x, 2 * x + 1], axis=2).reshape(BATCH, 2 * HIST)
    x3 = jnp.pad(xw, ((0, 0), (0, WPR - 2 * HIST)))
    return _encoder(x3, packed)


# speedup vs baseline: 1.1340x; 1.1340x over previous
import functools

import jax
import jax.numpy as jnp
from jax import lax
from jax.experimental import pallas as pl
from jax.experimental.pallas import tpu as pltpu
from jax.experimental.pallas import tpu_sc as plsc

BATCH = 4096
HIST = 200
DIM = 64
SIZE = 1000000
NWORDS = 2 * SIZE
HPAD = 208
WPR = 2 * HPAD
NGRP = WPR // 16
NC = 2
NS = 16
NW = NC * NS
BPW = BATCH // NW
LANES = 16
GB = 2
IB = 4
OROWS = 8
OB = 2
RPG = 8
NACC = HIST // RPG
NCHUNK = BPW // OROWS
SEG = NWORDS // NS


def _encoder_body(x_hbm, tbl_hbm, out_hbm, idx_v, rows_v, out_v, spmem_tbl,
                  isem0, isem1, isem2, isem3, gsem0, gsem1, osem0, osem1):
    cid = lax.axis_index("c")
    sid = lax.axis_index("s")
    wid = sid * NC + cid
    base = wid * BPW

    isems = (isem0, isem1, isem2, isem3)
    gsems = (gsem0, gsem1)
    osems = (osem0, osem1)

    pltpu.sync_copy(tbl_hbm.at[pl.ds(sid * SEG, SEG)],
                    spmem_tbl.at[pl.ds(sid * SEG, SEG)])
    plsc.subcore_barrier()

    def start_idx(b, islot):
        pltpu.async_copy(x_hbm.at[pl.ds(base + b, 1)],
                         idx_v.at[pl.ds(islot, 1)], isems[islot])

    def wait_idx(islot):
        pltpu.make_async_copy(x_hbm.at[pl.ds(0, 1)],
                              idx_v.at[pl.ds(islot, 1)],
                              isems[islot]).wait()

    def start_gather(islot, buf):
        for j in range(NGRP):
            idx_vec = idx_v[islot, 0, pl.ds(j * LANES, LANES)]
            pltpu.async_copy(
                spmem_tbl.at[idx_vec],
                rows_v.at[buf, pl.ds(j * LANES, LANES)], gsems[buf])

    def drain_gather(buf):
        pltpu.make_async_copy(
            spmem_tbl.at[pl.ds(0, WPR)], rows_v.at[buf], gsems[buf]).wait()

    def flush_out(chunk, oslot):
        pltpu.async_copy(out_v.at[oslot],
                         out_hbm.at[pl.ds(base + chunk * OROWS, OROWS)],
                         osems[oslot])

    def drain_out(oslot):
        pltpu.make_async_copy(out_v.at[oslot],
                              out_hbm.at[pl.ds(0, OROWS)],
                              osems[oslot]).wait()

    iota = lax.iota(jnp.int32, LANES)
    one_bit = jnp.ones((LANES,), jnp.int32)

    def accumulate(buf, oslot, orow):
        zero = jnp.zeros((LANES,), jnp.int32)

        def group(g, acc):
            v = rows_v[buf, pl.ds(g * 2 * RPG, 2 * RPG)]
            a0, a1, a2, a3 = acc
            for u in range(RPG):
                w0 = jnp.broadcast_to(v[2 * u], (LANES,))
                w1 = jnp.broadcast_to(v[2 * u + 1], (LANES,))
                a0 = a0 + ((w0 >> iota) & one_bit)
                a1 = a1 + ((w0 >> (iota + 16)) & one_bit)
                a2 = a2 + ((w1 >> iota) & one_bit)
                a3 = a3 + ((w1 >> (iota + 16)) & one_bit)
            return a0, a1, a2, a3

        acc = lax.fori_loop(0, NACC, group, (zero,) * 4)

        half = jnp.full((LANES,), HIST // 2, jnp.int32)
        onef = jnp.full((LANES,), 1.0, jnp.float32)
        for k in range(4):
            out_v[oslot, orow, pl.ds(k * LANES, LANES)] = jnp.where(
                acc[k] > half, onef, -onef)

    for b in range(IB):
        start_idx(b, b)
    for b in range(GB):
        wait_idx(b)
        start_gather(b, b)

    def outer(i, _):
        for cpar in range(OB):
            chunk = OB * i + cpar
            row0 = chunk * OROWS

            @pl.when(chunk >= OB)
            def _():
                drain_out(cpar)

            for u in range(OROWS):
                b = row0 + u
                drain_gather(u % GB)
                accumulate(u % GB, cpar, u)

                @pl.when(b + GB < BPW)
                def _():
                    wait_idx((u + GB) % IB)
                    start_gather((u + GB) % IB, u % GB)

                    @pl.when(b + IB < BPW)
                    def _():
                        start_idx(b + IB, u % IB)

            flush_out(chunk, cpar)
        return 0

    lax.fori_loop(0, NCHUNK // OB, outer, 0)

    for oslot in range(OB):
        drain_out(oslot)


@jax.jit
def _encoder(x3, packed):
    mesh = plsc.VectorSubcoreMesh(
        core_axis_name="c", subcore_axis_name="s", num_cores=NC,
        num_subcores=NS)
    return pl.kernel(
        _encoder_body,
        out_type=jax.ShapeDtypeStruct((BATCH, DIM), jnp.float32),
        mesh=mesh,
        scratch_types=[
            pltpu.VMEM((IB, 1, WPR), jnp.int32),
            pltpu.VMEM((GB, WPR), jnp.int32),
            pltpu.VMEM((OB, OROWS, DIM), jnp.float32),
            pltpu.VMEM_SHARED((NWORDS,), jnp.int32),
        ] + [pltpu.SemaphoreType.DMA] * (IB + GB + OB),
        compiler_params=pltpu.CompilerParams(use_tc_tiling_on_sc=False),
    )(x3, packed)


def kernel(x, embed_weight):
    bits = (embed_weight > 0).astype(jnp.int32)
    shifts = jnp.arange(32, dtype=jnp.int32)
    w0 = (bits[:, :32] << shifts).sum(axis=1, dtype=jnp.int32)
    w1 = (bits[:, 32:] << shifts).sum(axis=1, dtype=jnp.int32)
    packed = jnp.stack([w0, w1], axis=1).reshape(NWORDS)
    xw = jnp.stack([2 * x, 2 * x + 1], axis=2).reshape(BATCH, 2 * HIST)
    x3 = jnp.pad(xw, ((0, 0), (0, WPR - 2 * HIST))).reshape(BATCH, 1, WPR)
    return _encoder(x3, packed)


# speedup vs baseline: 1.4681x; 1.2946x over previous
import functools

import jax
import jax.numpy as jnp
from jax import lax
from jax.experimental import pallas as pl
from jax.experimental.pallas import tpu as pltpu
from jax.experimental.pallas import tpu_sc as plsc

BATCH = 4096
HIST = 200
DIM = 64
HPAD = 208
HALF = HPAD // 2
NC = 2
NS = 16
NW = NC * NS
BPW = BATCH // NW
LANES = 16
NV = DIM // LANES


NBUF = 4
UNROLL = 4


def _encoder_body(x_hbm, table_hbm, out_hbm, idx_v, rows_v, out_v, *sems):
    wid = lax.axis_index("s") * NC + lax.axis_index("c")
    base = wid * BPW

    pltpu.sync_copy(x_hbm.at[pl.ds(base, BPW)], idx_v)

    def start_gather(b, buf):
        for j in range(HPAD // LANES):
            idx_vec = idx_v[b, j]
            pltpu.async_copy(
                table_hbm.at[idx_vec],
                rows_v.at[buf, pl.ds(j * LANES, LANES)], sems[buf])

    def drain(buf):
        pltpu.make_async_copy(
            table_hbm.at[pl.ds(0, HPAD)], rows_v.at[buf], sems[buf]).wait()

    def accumulate(b, buf):
        zero = jnp.zeros((LANES,), jnp.float32)

        def body(j, acc):
            j0 = j * UNROLL
            r = [[rows_v[buf, j0 + u, pl.ds(k * LANES, LANES)]
                  for u in range(UNROLL)] for k in range(NV)]
            return tuple(
                acc[k] + ((r[k][0] + r[k][1]) + (r[k][2] + r[k][3]))
                for k in range(NV))

        acc = lax.fori_loop(0, HIST // UNROLL, body, (zero,) * NV)
        one = jnp.full((LANES,), 1.0, jnp.float32)
        for k in range(NV):
            out_v[b, pl.ds(k * LANES, LANES)] = jnp.where(
                acc[k] > 0.0, one, -one)

    for b in range(NBUF):
        start_gather(b, b)

    def outer(i, _):
        row0 = NBUF * i
        for b in range(NBUF):
            drain(b)
            accumulate(row0 + b, b)

            @pl.when(row0 + b + NBUF < BPW)
            def _():
                start_gather(row0 + b + NBUF, b)
        return 0

    lax.fori_loop(0, BPW // NBUF, outer, 0)

    pltpu.sync_copy(out_v, out_hbm.at[pl.ds(base, BPW)])


@jax.jit
def _encoder(x3, embed_weight):
    mesh = plsc.VectorSubcoreMesh(
        core_axis_name="c", subcore_axis_name="s", num_cores=NC,
        num_subcores=NS)
    return pl.kernel(
        _encoder_body,
        out_type=jax.ShapeDtypeStruct((BATCH, DIM), jnp.float32),
        mesh=mesh,
        scratch_types=[
            pltpu.VMEM((BPW, HPAD // LANES, LANES), jnp.int32),
            pltpu.VMEM((NBUF, HPAD, DIM), jnp.float32),
            pltpu.VMEM((BPW, DIM), jnp.float32),
        ] + [pltpu.SemaphoreType.DMA] * NBUF,
        compiler_params=pltpu.CompilerParams(use_tc_tiling_on_sc=False),
    )(x3, embed_weight)


def kernel(x, embed_weight):
    x3 = jnp.pad(x, ((0, 0), (0, HPAD - HIST))).reshape(
        BATCH, HPAD // LANES, LANES)
    return _encoder(x3, embed_weight)
